# Initial kernel scaffold; baseline (speedup 1.0000x reference)
import jax
import jax.numpy as jnp
from jax.experimental import pallas as pl


def kernel(hidden_states, kernel, bias):
    raise NotImplementedError("write your pallas kernel here")


# speedup vs baseline: 4.5859x; 4.5859x over previous
import functools

import jax
import jax.numpy as jnp
from jax.experimental import pallas as pl
from jax.experimental.pallas import tpu as pltpu

_TOP_K = 8
_NUM_EXPERTS = 64
_ROW_BLOCK = 512


def _router_block(hs_ref, w_ref, b_ref, scores_ref, idx_ref):
    logits = (
        jnp.dot(hs_ref[...], w_ref[...], preferred_element_type=jnp.float32)
        + b_ref[...]
    )
    rows = logits.shape[0]
    col = jax.lax.broadcasted_iota(jnp.int32, (rows, _NUM_EXPERTS), 1)
    neg_inf = jnp.float32(-jnp.inf)

    work = logits
    vals = []
    idxs = []
    for _ in range(_TOP_K):
        m = jnp.max(work, axis=1, keepdims=True)
        eq = work == m
        idx = jnp.min(jnp.where(eq, col, _NUM_EXPERTS), axis=1, keepdims=True)
        vals.append(m)
        idxs.append(idx)
        work = jnp.where(col == idx, neg_inf, work)

    exps = [jnp.exp(v - vals[0]) for v in vals]
    denom = exps[0]
    for e in exps[1:]:
        denom = denom + e
    inv = 1.0 / denom

    scores = jnp.zeros((rows, _NUM_EXPERTS), dtype=jnp.float32)
    for k in range(_TOP_K):
        scores = jnp.where(col == idxs[k], exps[k] * inv, scores)
    scores_ref[...] = scores
    idx_ref[...] = jnp.concatenate(idxs, axis=1).astype(jnp.int32)


def kernel(hidden_states, kernel, bias):
    hidden_dim = hidden_states.shape[-1]
    hs = hidden_states.reshape(-1, hidden_dim)
    n_rows = hs.shape[0]
    bias2d = bias.reshape(1, _NUM_EXPERTS)

    grid = (n_rows // _ROW_BLOCK,)
    scores, indices = pl.pallas_call(
        _router_block,
        grid=grid,
        in_specs=[
            pl.BlockSpec((_ROW_BLOCK, hidden_dim), lambda i: (i, 0)),
            pl.BlockSpec((hidden_dim, _NUM_EXPERTS), lambda i: (0, 0)),
            pl.BlockSpec((1, _NUM_EXPERTS), lambda i: (0, 0)),
        ],
        out_specs=[
            pl.BlockSpec((_ROW_BLOCK, _NUM_EXPERTS), lambda i: (i, 0)),
            pl.BlockSpec((_ROW_BLOCK, _TOP_K), lambda i: (i, 0)),
        ],
        out_shape=[
            jax.ShapeDtypeStruct((n_rows, _NUM_EXPERTS), jnp.float32),
            jax.ShapeDtypeStruct((n_rows, _TOP_K), jnp.int32),
        ],
        compiler_params=pltpu.CompilerParams(
            dimension_semantics=("arbitrary",),
        ),
    )(hs, kernel, bias2d)
    return scores, indices
